# baseline (speedup 1.0000x reference)
import functools

import jax
import jax.numpy as jnp
from jax import lax
from jax.experimental import pallas as pl
from jax.experimental.pallas import tpu as pltpu
from jax.experimental.pallas import tpu_sc as plsc

N = 10000
E = 320000
D = 128
NCLS = 40
NUM_LAYERS = 7
EPS = 1e-7

ROWS = 1000
GRID = N // ROWS

NS = 16
CHUNK = 96
EDGES_PER_TILE = 20160
E_PAD = NS * EDGES_PER_TILE
NCHUNK = EDGES_PER_TILE // CHUNK
SUPER = 42
NSUPER = NCHUNK // SUPER
PADROWS = 488
TROWS = 632
TROWS_LAST = N - TROWS * (NS - 1)


def _pack_p(h, P_ref):
    m = jnp.maximum(h, 0.0) + EPS
    g = jnp.exp(m)
    P_ref[0] = g
    P_ref[1] = g * m


def _enc_body(x_ref, W_ref, b_ref, h_ref, P_ref):
    h = jnp.dot(x_ref[...], W_ref[...], preferred_element_type=jnp.float32)
    h = h + b_ref[...]
    h_ref[...] = h
    _pack_p(h, P_ref)


def _layer_norm_relu(h, g, b):
    mu = jnp.mean(h, axis=-1, keepdims=True)
    var = jnp.mean((h - mu) * (h - mu), axis=-1, keepdims=True)
    return jnp.maximum((h - mu) / jnp.sqrt(var + 1e-5) * g + b, 0.0)


def _mid_body(add_res, S_ref, xp_ref, hp_ref, W_ref, b_ref, g_ref, be_ref,
              h_ref, xn_ref, P_ref):
    agg = S_ref[1] / (S_ref[0] + 1e-16)
    h = jnp.dot(xp_ref[...] + agg, W_ref[...],
                preferred_element_type=jnp.float32) + b_ref[...]
    if add_res:
        h = h + hp_ref[...]
    h_ref[...] = h
    xn = _layer_norm_relu(h, g_ref[...], be_ref[...])
    xn_ref[...] = xn
    _pack_p(xn, P_ref)


def _fin_body(S_ref, xp_ref, hp_ref, W_ref, b_ref, g_ref, be_ref,
              Wp_ref, bp_ref, out_ref):
    agg = S_ref[1] / (S_ref[0] + 1e-16)
    h = jnp.dot(xp_ref[...] + agg, W_ref[...],
                preferred_element_type=jnp.float32) + b_ref[...]
    h = h + hp_ref[...]
    u = _layer_norm_relu(h, g_ref[...], be_ref[...])
    logits = jnp.dot(u, Wp_ref[...],
                     preferred_element_type=jnp.float32) + bp_ref[...]
    mx = jnp.max(logits, axis=-1, keepdims=True)
    z = logits - mx
    lse = jnp.log(jnp.sum(jnp.exp(z), axis=-1, keepdims=True))
    out_ref[...] = z - lse


_row_spec = pl.BlockSpec((ROWS, D), lambda i: (i, 0))
_full_spec = pl.BlockSpec((D, D), lambda i: (0, 0))
_vec_spec = pl.BlockSpec((1, D), lambda i: (0, 0))
_P_spec = pl.BlockSpec((2, ROWS, D), lambda i: (0, i, 0))

_fD = jax.ShapeDtypeStruct((N, D), jnp.float32)
_fP = jax.ShapeDtypeStruct((2, N, D), jnp.float32)

_enc_call = pl.pallas_call(
    _enc_body,
    grid=(GRID,),
    in_specs=[_row_spec, _full_spec, _vec_spec],
    out_specs=[_row_spec, _P_spec],
    out_shape=[_fD, _fP],
)

_mid_specs = dict(
    grid=(GRID,),
    in_specs=[_P_spec, _row_spec, _row_spec, _full_spec, _vec_spec,
              _vec_spec, _vec_spec],
    out_specs=[_row_spec, _row_spec, _P_spec],
    out_shape=[_fD, _fD, _fP],
)
_mid_call_nores = pl.pallas_call(functools.partial(_mid_body, False), **_mid_specs)
_mid_call_res = pl.pallas_call(functools.partial(_mid_body, True), **_mid_specs)

_fin_call = pl.pallas_call(
    _fin_body,
    grid=(GRID,),
    in_specs=[_P_spec, _row_spec, _row_spec, _full_spec, _vec_spec,
              _vec_spec, _vec_spec,
              pl.BlockSpec((D, NCLS), lambda i: (0, 0)),
              pl.BlockSpec((1, NCLS), lambda i: (0, 0))],
    out_specs=pl.BlockSpec((ROWS, NCLS), lambda i: (i, 0)),
    out_shape=jax.ShapeDtypeStruct((N, NCLS), jnp.float32),
)


_sc_mesh = plsc.VectorSubcoreMesh(core_axis_name="c", subcore_axis_name="s")


@functools.partial(
    pl.kernel,
    mesh=_sc_mesh,
    out_type=jax.ShapeDtypeStruct((2 * N, D), jnp.float32),
    scratch_types=[
        pltpu.VMEM((SUPER * CHUNK,), jnp.int32),
        pltpu.VMEM((SUPER, CHUNK), jnp.int32),
        pltpu.VMEM((CHUNK, D), jnp.float32),
        pltpu.VMEM((CHUNK, D), jnp.float32),
        pltpu.VMEM((CHUNK, D), jnp.float32),
        pltpu.VMEM_SHARED((N + PADROWS, D), jnp.float32),
        pltpu.SemaphoreType.DMA,
        pltpu.SemaphoreType.DMA,
        pltpu.SemaphoreType.DMA,
        pltpu.SemaphoreType.DMA,
        pltpu.SemaphoreType.DMA,
        pltpu.SemaphoreType.DMA,
    ],
)
def _spmm_sc(P_hbm, src_hbm, dst_hbm, zeros_hbm, out_hbm,
             src_v, dst_v, rows_a, rows_b, rows_c, S_sp, sem_a, sem_b, sem_c,
             sem_sa, sem_sb, sem_sc):
    c = lax.axis_index("c")
    s = lax.axis_index("s")

    r0 = s * TROWS
    coff = c * N
    ebase = (c * NS + s) * EDGES_PER_TILE

    @pl.when(s < NS - 1)
    def _zero_main():
        pltpu.sync_copy(zeros_hbm, S_sp.at[pl.ds(r0, TROWS)])

    @pl.when(s == NS - 1)
    def _zero_last():
        pltpu.sync_copy(zeros_hbm.at[pl.ds(0, TROWS_LAST)],
                        S_sp.at[pl.ds((NS - 1) * TROWS, TROWS_LAST)])

    plsc.subcore_barrier()

    def _gather(i, buf, sem):
        pltpu.async_copy(P_hbm.at[src_v.at[pl.ds(i * CHUNK, CHUNK)]],
                         buf, sem)

    def _gwait(i, buf, sem):
        pltpu.make_async_copy(P_hbm.at[src_v.at[pl.ds(i * CHUNK, CHUNK)]],
                              buf, sem).wait()

    def _scat(i, buf, sem):
        pltpu.async_copy(buf, S_sp.at[dst_v.at[i]], sem, add=True)

    def _swait(i, buf, sem):
        pltpu.make_async_copy(buf, S_sp.at[dst_v.at[i]], sem).wait()

    def super_body(u, carry):
        pltpu.sync_copy(
            src_hbm.at[pl.ds(ebase + u * (SUPER * CHUNK), SUPER * CHUNK)],
            src_v)
        pltpu.sync_copy(dst_hbm.at[s, u], dst_v)
        _gather(0, rows_a, sem_a)
        _gather(1, rows_b, sem_b)
        _gather(2, rows_c, sem_c)

        bufs = None

        def body(k, carry2):
            i0 = 3 * k
            refire = k < SUPER // 3 - 1
            trip = [(rows_a, sem_a, sem_sa), (rows_b, sem_b, sem_sb),
                    (rows_c, sem_c, sem_sc)]
            for d, (buf, gsem, ssem) in enumerate(trip):
                _gwait(i0 + d, buf, gsem)
                _scat(i0 + d, buf, ssem)
            for d, (buf, gsem, ssem) in enumerate(trip):
                @pl.when(refire)
                def _next(buf=buf, gsem=gsem, ssem=ssem, d=d):
                    _swait(i0 + d, buf, ssem)
                    _gather(i0 + d + 3, buf, gsem)
            return carry2

        lax.fori_loop(0, SUPER // 3, body, 0)
        for d, (buf, ssem) in enumerate(
                [(rows_a, sem_sa), (rows_b, sem_sb), (rows_c, sem_sc)]):
            _swait(SUPER - 3 + d, buf, ssem)
        return carry

    lax.fori_loop(0, NSUPER, super_body, 0)
    plsc.subcore_barrier()

    @pl.when(s < NS - 1)
    def _wb_main():
        pltpu.sync_copy(S_sp.at[pl.ds(r0, TROWS)],
                        out_hbm.at[pl.ds(coff + r0, TROWS)])

    @pl.when(s == NS - 1)
    def _wb_last():
        pltpu.sync_copy(S_sp.at[pl.ds((NS - 1) * TROWS, TROWS_LAST)],
                        out_hbm.at[pl.ds(coff + (NS - 1) * TROWS, TROWS_LAST)])


def kernel(x, edge_index, W_enc, b_enc, W_gcn, b_gcn, gamma, beta,
           W_pred, b_pred):
    src = edge_index[0].astype(jnp.int32)
    dst = edge_index[1].astype(jnp.int32)
    srcp = jnp.concatenate(
        [src, jnp.arange(E_PAD - E, dtype=jnp.int32) % N])
    dstp = jnp.concatenate(
        [dst, N + (jnp.arange(E_PAD - E, dtype=jnp.int32) % PADROWS)])
    src2 = jnp.concatenate([srcp, srcp + N])
    dst3 = dstp.reshape(NS, NSUPER, SUPER, CHUNK)
    zeros = jnp.zeros((TROWS, D), jnp.float32)

    def spmm(P):
        flat = _spmm_sc(P.reshape(2 * N, D), src2, dst3, zeros)
        return flat.reshape(2, N, D)

    b_enc2 = b_enc.reshape(1, D)
    h, P = _enc_call(x, W_enc, b_enc2)
    xp = h
    for r in range(NUM_LAYERS - 1):
        S = spmm(P)
        call = _mid_call_nores if r == 0 else _mid_call_res
        h, xp, P = call(S, xp, h, W_gcn[r], b_gcn[r].reshape(1, D),
                        gamma[r].reshape(1, D), beta[r].reshape(1, D))
    S = spmm(P)
    r = NUM_LAYERS - 1
    return _fin_call(S, xp, h, W_gcn[r], b_gcn[r].reshape(1, D),
                     gamma[r].reshape(1, D), beta[r].reshape(1, D),
                     W_pred, b_pred.reshape(1, NCLS))


# speedup vs baseline: 1.2414x; 1.2414x over previous
import functools

import jax
import jax.numpy as jnp
from jax import lax
from jax.experimental import pallas as pl
from jax.experimental.pallas import tpu as pltpu
from jax.experimental.pallas import tpu_sc as plsc

N = 10000
E = 320000
D = 128
NCLS = 40
NUM_LAYERS = 7
EPS = 1e-7

ROWS = 1000
GRID = N // ROWS

NS = 16
CHUNK = 96
EDGES_PER_TILE = 20160
E_PAD = NS * EDGES_PER_TILE
NCHUNK = EDGES_PER_TILE // CHUNK
SUPER = 42
NSUPER = NCHUNK // SUPER
PADROWS = 488
TROWS = 632
TROWS_LAST = N - TROWS * (NS - 1)


def _pack_p(h, P_ref):
    m = jnp.maximum(h, 0.0) + EPS
    g = jnp.exp(m)
    P_ref[0] = g
    P_ref[1] = g * m


def _enc_body(x_ref, W_ref, b_ref, h_ref, P_ref):
    h = jnp.dot(x_ref[...], W_ref[...], preferred_element_type=jnp.float32)
    h = h + b_ref[...]
    h_ref[...] = h
    _pack_p(h, P_ref)


def _layer_norm_relu(h, g, b):
    mu = jnp.mean(h, axis=-1, keepdims=True)
    var = jnp.mean((h - mu) * (h - mu), axis=-1, keepdims=True)
    return jnp.maximum((h - mu) / jnp.sqrt(var + 1e-5) * g + b, 0.0)


def _mid_body(add_res, S_ref, xp_ref, hp_ref, W_ref, b_ref, g_ref, be_ref,
              h_ref, xn_ref, P_ref):
    agg = S_ref[1] / (S_ref[0] + 1e-16)
    h = jnp.dot(xp_ref[...] + agg, W_ref[...],
                preferred_element_type=jnp.float32) + b_ref[...]
    if add_res:
        h = h + hp_ref[...]
    h_ref[...] = h
    xn = _layer_norm_relu(h, g_ref[...], be_ref[...])
    xn_ref[...] = xn
    _pack_p(xn, P_ref)


def _fin_body(S_ref, xp_ref, hp_ref, W_ref, b_ref, g_ref, be_ref,
              Wp_ref, bp_ref, out_ref):
    agg = S_ref[1] / (S_ref[0] + 1e-16)
    h = jnp.dot(xp_ref[...] + agg, W_ref[...],
                preferred_element_type=jnp.float32) + b_ref[...]
    h = h + hp_ref[...]
    u = _layer_norm_relu(h, g_ref[...], be_ref[...])
    logits = jnp.dot(u, Wp_ref[...],
                     preferred_element_type=jnp.float32) + bp_ref[...]
    mx = jnp.max(logits, axis=-1, keepdims=True)
    z = logits - mx
    lse = jnp.log(jnp.sum(jnp.exp(z), axis=-1, keepdims=True))
    out_ref[...] = z - lse


_row_spec = pl.BlockSpec((ROWS, D), lambda i: (i, 0))
_full_spec = pl.BlockSpec((D, D), lambda i: (0, 0))
_vec_spec = pl.BlockSpec((1, D), lambda i: (0, 0))
_P_spec = pl.BlockSpec((2, ROWS, D), lambda i: (0, i, 0))

_fD = jax.ShapeDtypeStruct((N, D), jnp.float32)
_fP = jax.ShapeDtypeStruct((2, N, D), jnp.float32)

_enc_call = pl.pallas_call(
    _enc_body,
    grid=(GRID,),
    in_specs=[_row_spec, _full_spec, _vec_spec],
    out_specs=[_row_spec, _P_spec],
    out_shape=[_fD, _fP],
)

_mid_specs = dict(
    grid=(GRID,),
    in_specs=[_P_spec, _row_spec, _row_spec, _full_spec, _vec_spec,
              _vec_spec, _vec_spec],
    out_specs=[_row_spec, _row_spec, _P_spec],
    out_shape=[_fD, _fD, _fP],
)
_mid_call_nores = pl.pallas_call(functools.partial(_mid_body, False), **_mid_specs)
_mid_call_res = pl.pallas_call(functools.partial(_mid_body, True), **_mid_specs)

_fin_call = pl.pallas_call(
    _fin_body,
    grid=(GRID,),
    in_specs=[_P_spec, _row_spec, _row_spec, _full_spec, _vec_spec,
              _vec_spec, _vec_spec,
              pl.BlockSpec((D, NCLS), lambda i: (0, 0)),
              pl.BlockSpec((1, NCLS), lambda i: (0, 0))],
    out_specs=pl.BlockSpec((ROWS, NCLS), lambda i: (i, 0)),
    out_shape=jax.ShapeDtypeStruct((N, NCLS), jnp.float32),
)


_sc_mesh = plsc.VectorSubcoreMesh(core_axis_name="c", subcore_axis_name="s")


@functools.partial(
    pl.kernel,
    mesh=_sc_mesh,
    out_type=jax.ShapeDtypeStruct((2 * N, D), jnp.float32),
    scratch_types=[
        pltpu.VMEM((SUPER * CHUNK,), jnp.int32),
        pltpu.VMEM((SUPER, CHUNK), jnp.int32),
        pltpu.VMEM((CHUNK, D), jnp.float32),
        pltpu.VMEM((CHUNK, D), jnp.float32),
        pltpu.VMEM((CHUNK, D), jnp.float32),
        pltpu.VMEM_SHARED((N + PADROWS, D), jnp.float32),
        pltpu.SemaphoreType.DMA,
        pltpu.SemaphoreType.DMA,
        pltpu.SemaphoreType.DMA,
    ],
)
def _spmm_sc(P_hbm, src_hbm, dst_hbm, zeros_hbm, out_hbm,
             src_v, dst_v, rows_a, rows_b, rows_c, S_sp, sem_a, sem_b, sem_c):
    c = lax.axis_index("c")
    s = lax.axis_index("s")

    r0 = s * TROWS
    coff = c * N
    ebase = (c * NS + s) * EDGES_PER_TILE

    @pl.when(s < NS - 1)
    def _zero_main():
        pltpu.sync_copy(zeros_hbm, S_sp.at[pl.ds(r0, TROWS)])

    @pl.when(s == NS - 1)
    def _zero_last():
        pltpu.sync_copy(zeros_hbm.at[pl.ds(0, TROWS_LAST)],
                        S_sp.at[pl.ds((NS - 1) * TROWS, TROWS_LAST)])

    plsc.subcore_barrier()

    def _gather(i, buf, sem):
        pltpu.async_copy(P_hbm.at[src_v.at[pl.ds(i * CHUNK, CHUNK)]],
                         buf, sem)

    def _gwait(i, buf, sem):
        pltpu.make_async_copy(P_hbm.at[src_v.at[pl.ds(i * CHUNK, CHUNK)]],
                              buf, sem).wait()

    def _scat(i, buf):
        pltpu.sync_copy(buf, S_sp.at[dst_v.at[i]], add=True)

    def super_body(u, carry):
        pltpu.sync_copy(
            src_hbm.at[pl.ds(ebase + u * (SUPER * CHUNK), SUPER * CHUNK)],
            src_v)
        pltpu.sync_copy(dst_hbm.at[s, u], dst_v)
        _gather(0, rows_a, sem_a)
        _gather(1, rows_b, sem_b)
        _gather(2, rows_c, sem_c)

        def body(k, carry2):
            i0 = 3 * k
            refire = k < SUPER // 3 - 1
            for d, (buf, sem) in enumerate(
                    [(rows_a, sem_a), (rows_b, sem_b), (rows_c, sem_c)]):
                _gwait(i0 + d, buf, sem)
                _scat(i0 + d, buf)

                @pl.when(refire)
                def _next(buf=buf, sem=sem, d=d):
                    _gather(i0 + d + 3, buf, sem)
            return carry2

        lax.fori_loop(0, SUPER // 3, body, 0)
        return carry

    lax.fori_loop(0, NSUPER, super_body, 0)
    plsc.subcore_barrier()

    @pl.when(s < NS - 1)
    def _wb_main():
        pltpu.sync_copy(S_sp.at[pl.ds(r0, TROWS)],
                        out_hbm.at[pl.ds(coff + r0, TROWS)])

    @pl.when(s == NS - 1)
    def _wb_last():
        pltpu.sync_copy(S_sp.at[pl.ds((NS - 1) * TROWS, TROWS_LAST)],
                        out_hbm.at[pl.ds(coff + (NS - 1) * TROWS, TROWS_LAST)])


def kernel(x, edge_index, W_enc, b_enc, W_gcn, b_gcn, gamma, beta,
           W_pred, b_pred):
    src = edge_index[0].astype(jnp.int32)
    dst = edge_index[1].astype(jnp.int32)
    srcp = jnp.concatenate(
        [src, jnp.arange(E_PAD - E, dtype=jnp.int32) % N])
    dstp = jnp.concatenate(
        [dst, N + (jnp.arange(E_PAD - E, dtype=jnp.int32) % PADROWS)])
    src2 = jnp.concatenate([srcp, srcp + N])
    dst3 = dstp.reshape(NS, NSUPER, SUPER, CHUNK)
    zeros = jnp.zeros((TROWS, D), jnp.float32)

    def spmm(P):
        flat = _spmm_sc(P.reshape(2 * N, D), src2, dst3, zeros)
        return flat.reshape(2, N, D)

    b_enc2 = b_enc.reshape(1, D)
    h, P = _enc_call(x, W_enc, b_enc2)
    xp = h
    for r in range(NUM_LAYERS - 1):
        S = spmm(P)
        call = _mid_call_nores if r == 0 else _mid_call_res
        h, xp, P = call(S, xp, h, W_gcn[r], b_gcn[r].reshape(1, D),
                        gamma[r].reshape(1, D), beta[r].reshape(1, D))
    S = spmm(P)
    r = NUM_LAYERS - 1
    return _fin_call(S, xp, h, W_gcn[r], b_gcn[r].reshape(1, D),
                     gamma[r].reshape(1, D), beta[r].reshape(1, D),
                     W_pred, b_pred.reshape(1, NCLS))


# speedup vs baseline: 1.2460x; 1.0037x over previous
import functools

import jax
import jax.numpy as jnp
from jax import lax
from jax.experimental import pallas as pl
from jax.experimental.pallas import tpu as pltpu
from jax.experimental.pallas import tpu_sc as plsc

N = 10000
E = 320000
D = 128
NCLS = 40
NUM_LAYERS = 7
EPS = 1e-7

ROWS = 1000
GRID = N // ROWS

NS = 16
CHUNK = 96
EDGES_PER_TILE = 20160
E_PAD = NS * EDGES_PER_TILE
NCHUNK = EDGES_PER_TILE // CHUNK
SUPER = 42
NSUPER = NCHUNK // SUPER
PADROWS = 488
TROWS = 632
TROWS_LAST = N - TROWS * (NS - 1)


def _pack_p(h, P_ref):
    m = jnp.maximum(h, 0.0) + EPS
    g = jnp.exp(m)
    P_ref[0] = g
    P_ref[1] = g * m


def _enc_body(x_ref, W_ref, b_ref, h_ref, P_ref):
    h = jnp.dot(x_ref[...], W_ref[...], preferred_element_type=jnp.float32)
    h = h + b_ref[...]
    h_ref[...] = h
    _pack_p(h, P_ref)


def _layer_norm_relu(h, g, b):
    mu = jnp.mean(h, axis=-1, keepdims=True)
    var = jnp.mean((h - mu) * (h - mu), axis=-1, keepdims=True)
    return jnp.maximum((h - mu) / jnp.sqrt(var + 1e-5) * g + b, 0.0)


def _mid_body(add_res, S_ref, xp_ref, hp_ref, W_ref, b_ref, g_ref, be_ref,
              h_ref, xn_ref, P_ref):
    agg = S_ref[1] / (S_ref[0] + 1e-16)
    h = jnp.dot(xp_ref[...] + agg, W_ref[...],
                preferred_element_type=jnp.float32) + b_ref[...]
    if add_res:
        h = h + hp_ref[...]
    h_ref[...] = h
    xn = _layer_norm_relu(h, g_ref[...], be_ref[...])
    xn_ref[...] = xn
    _pack_p(xn, P_ref)


def _fin_body(S_ref, xp_ref, hp_ref, W_ref, b_ref, g_ref, be_ref,
              Wp_ref, bp_ref, out_ref):
    agg = S_ref[1] / (S_ref[0] + 1e-16)
    h = jnp.dot(xp_ref[...] + agg, W_ref[...],
                preferred_element_type=jnp.float32) + b_ref[...]
    h = h + hp_ref[...]
    u = _layer_norm_relu(h, g_ref[...], be_ref[...])
    logits = jnp.dot(u, Wp_ref[...],
                     preferred_element_type=jnp.float32) + bp_ref[...]
    mx = jnp.max(logits, axis=-1, keepdims=True)
    z = logits - mx
    lse = jnp.log(jnp.sum(jnp.exp(z), axis=-1, keepdims=True))
    out_ref[...] = z - lse


_row_spec = pl.BlockSpec((ROWS, D), lambda i: (i, 0))
_full_spec = pl.BlockSpec((D, D), lambda i: (0, 0))
_vec_spec = pl.BlockSpec((1, D), lambda i: (0, 0))
_P_spec = pl.BlockSpec((2, ROWS, D), lambda i: (0, i, 0))

_fD = jax.ShapeDtypeStruct((N, D), jnp.float32)
_fP = jax.ShapeDtypeStruct((2, N, D), jnp.float32)

_enc_call = pl.pallas_call(
    _enc_body,
    grid=(GRID,),
    in_specs=[_row_spec, _full_spec, _vec_spec],
    out_specs=[_row_spec, _P_spec],
    out_shape=[_fD, _fP],
)

_mid_specs = dict(
    grid=(GRID,),
    in_specs=[_P_spec, _row_spec, _row_spec, _full_spec, _vec_spec,
              _vec_spec, _vec_spec],
    out_specs=[_row_spec, _row_spec, _P_spec],
    out_shape=[_fD, _fD, _fP],
)
_mid_call_nores = pl.pallas_call(functools.partial(_mid_body, False), **_mid_specs)
_mid_call_res = pl.pallas_call(functools.partial(_mid_body, True), **_mid_specs)

_fin_call = pl.pallas_call(
    _fin_body,
    grid=(GRID,),
    in_specs=[_P_spec, _row_spec, _row_spec, _full_spec, _vec_spec,
              _vec_spec, _vec_spec,
              pl.BlockSpec((D, NCLS), lambda i: (0, 0)),
              pl.BlockSpec((1, NCLS), lambda i: (0, 0))],
    out_specs=pl.BlockSpec((ROWS, NCLS), lambda i: (i, 0)),
    out_shape=jax.ShapeDtypeStruct((N, NCLS), jnp.float32),
)


_sc_mesh = plsc.VectorSubcoreMesh(core_axis_name="c", subcore_axis_name="s")


@functools.partial(
    pl.kernel,
    mesh=_sc_mesh,
    out_type=jax.ShapeDtypeStruct((2 * N, D), jnp.float32),
    scratch_types=[
        pltpu.VMEM((SUPER * CHUNK,), jnp.int32),
        pltpu.VMEM((SUPER, CHUNK), jnp.int32),
        pltpu.VMEM((CHUNK, D), jnp.float32),
        pltpu.VMEM((CHUNK, D), jnp.float32),
        pltpu.VMEM((CHUNK, D), jnp.float32),
        pltpu.VMEM_SHARED((N + PADROWS, D), jnp.float32),
        pltpu.SemaphoreType.DMA,
        pltpu.SemaphoreType.DMA,
        pltpu.SemaphoreType.DMA,
    ],
)
def _spmm_sc(P_hbm, src_hbm, dst_hbm, zeros_hbm, out_hbm,
             src_v, dst_v, rows_a, rows_b, rows_c, S_sp, sem_a, sem_b, sem_c):
    c = lax.axis_index("c")
    s = lax.axis_index("s")

    r0 = s * TROWS
    coff = c * N
    ebase = (c * NS + s) * EDGES_PER_TILE

    @pl.when(s < NS - 1)
    def _zero_main():
        pltpu.sync_copy(zeros_hbm.at[pl.ds(r0, TROWS)],
                        S_sp.at[pl.ds(r0, TROWS)])

    @pl.when(s == NS - 1)
    def _zero_last():
        pltpu.sync_copy(zeros_hbm.at[pl.ds((NS - 1) * TROWS, TROWS_LAST)],
                        S_sp.at[pl.ds((NS - 1) * TROWS, TROWS_LAST)])

    plsc.subcore_barrier()

    def _gather(i, buf, sem):
        pltpu.async_copy(P_hbm.at[src_v.at[pl.ds(i * CHUNK, CHUNK)]],
                         buf, sem)

    def _gwait(i, buf, sem):
        pltpu.make_async_copy(P_hbm.at[src_v.at[pl.ds(i * CHUNK, CHUNK)]],
                              buf, sem).wait()

    def _scat(i, buf):
        pltpu.sync_copy(buf, S_sp.at[dst_v.at[i]], add=True)

    def super_body(u, carry):
        pltpu.sync_copy(
            src_hbm.at[pl.ds(ebase + u * (SUPER * CHUNK), SUPER * CHUNK)],
            src_v)
        pltpu.sync_copy(dst_hbm.at[s, u], dst_v)
        _gather(0, rows_a, sem_a)
        _gather(1, rows_b, sem_b)
        _gather(2, rows_c, sem_c)

        def body(k, carry2):
            i0 = 3 * k
            refire = k < SUPER // 3 - 1
            for d, (buf, sem) in enumerate(
                    [(rows_a, sem_a), (rows_b, sem_b), (rows_c, sem_c)]):
                _gwait(i0 + d, buf, sem)
                _scat(i0 + d, buf)

                @pl.when(refire)
                def _next(buf=buf, sem=sem, d=d):
                    _gather(i0 + d + 3, buf, sem)
            return carry2

        lax.fori_loop(0, SUPER // 3, body, 0)
        return carry

    lax.fori_loop(0, NSUPER, super_body, 0)
    plsc.subcore_barrier()

    @pl.when(s < NS - 1)
    def _wb_main():
        pltpu.sync_copy(S_sp.at[pl.ds(r0, TROWS)],
                        out_hbm.at[pl.ds(coff + r0, TROWS)])

    @pl.when(s == NS - 1)
    def _wb_last():
        pltpu.sync_copy(S_sp.at[pl.ds((NS - 1) * TROWS, TROWS_LAST)],
                        out_hbm.at[pl.ds(coff + (NS - 1) * TROWS, TROWS_LAST)])


def kernel(x, edge_index, W_enc, b_enc, W_gcn, b_gcn, gamma, beta,
           W_pred, b_pred):
    src = edge_index[0].astype(jnp.int32)
    dst = edge_index[1].astype(jnp.int32)
    srcp = jnp.concatenate(
        [src, jnp.arange(E_PAD - E, dtype=jnp.int32) % N])
    dstp = jnp.concatenate(
        [dst, N + (jnp.arange(E_PAD - E, dtype=jnp.int32) % PADROWS)])
    src2 = jnp.concatenate([srcp, srcp + N])
    dst3 = dstp.reshape(NS, NSUPER, SUPER, CHUNK)
    zeros = jnp.zeros((N, D), jnp.float32)

    def spmm(P):
        flat = _spmm_sc(P.reshape(2 * N, D), src2, dst3, zeros)
        return flat.reshape(2, N, D)

    b_enc2 = b_enc.reshape(1, D)
    h, P = _enc_call(x, W_enc, b_enc2)
    xp = h
    for r in range(NUM_LAYERS - 1):
        S = spmm(P)
        call = _mid_call_nores if r == 0 else _mid_call_res
        h, xp, P = call(S, xp, h, W_gcn[r], b_gcn[r].reshape(1, D),
                        gamma[r].reshape(1, D), beta[r].reshape(1, D))
    S = spmm(P)
    r = NUM_LAYERS - 1
    return _fin_call(S, xp, h, W_gcn[r], b_gcn[r].reshape(1, D),
                     gamma[r].reshape(1, D), beta[r].reshape(1, D),
                     W_pred, b_pred.reshape(1, NCLS))


# speedup vs baseline: 1.2462x; 1.0002x over previous
import functools

import jax
import jax.numpy as jnp
from jax import lax
from jax.experimental import pallas as pl
from jax.experimental.pallas import tpu as pltpu
from jax.experimental.pallas import tpu_sc as plsc

N = 10000
E = 320000
D = 128
NCLS = 40
NUM_LAYERS = 7
EPS = 1e-7

ROWS = 1000
GRID = N // ROWS

NS = 16
CHUNK = 64
EDGES_PER_TILE = 20480
E_PAD = NS * EDGES_PER_TILE
NCHUNK = EDGES_PER_TILE // CHUNK
SUPER = 64
NSUPER = NCHUNK // SUPER
PADROWS = 488
TROWS = 632
TROWS_LAST = N - TROWS * (NS - 1)


def _pack_p(h, P_ref):
    m = jnp.maximum(h, 0.0) + EPS
    g = jnp.exp(m)
    P_ref[0] = g
    P_ref[1] = g * m


def _enc_body(x_ref, W_ref, b_ref, h_ref, P_ref):
    h = jnp.dot(x_ref[...], W_ref[...], preferred_element_type=jnp.float32)
    h = h + b_ref[...]
    h_ref[...] = h
    _pack_p(h, P_ref)


def _layer_norm_relu(h, g, b):
    mu = jnp.mean(h, axis=-1, keepdims=True)
    var = jnp.mean((h - mu) * (h - mu), axis=-1, keepdims=True)
    return jnp.maximum((h - mu) / jnp.sqrt(var + 1e-5) * g + b, 0.0)


def _mid_body(add_res, S_ref, xp_ref, hp_ref, W_ref, b_ref, g_ref, be_ref,
              h_ref, xn_ref, P_ref):
    agg = S_ref[1] / (S_ref[0] + 1e-16)
    h = jnp.dot(xp_ref[...] + agg, W_ref[...],
                preferred_element_type=jnp.float32) + b_ref[...]
    if add_res:
        h = h + hp_ref[...]
    h_ref[...] = h
    xn = _layer_norm_relu(h, g_ref[...], be_ref[...])
    xn_ref[...] = xn
    _pack_p(xn, P_ref)


def _fin_body(S_ref, xp_ref, hp_ref, W_ref, b_ref, g_ref, be_ref,
              Wp_ref, bp_ref, out_ref):
    agg = S_ref[1] / (S_ref[0] + 1e-16)
    h = jnp.dot(xp_ref[...] + agg, W_ref[...],
                preferred_element_type=jnp.float32) + b_ref[...]
    h = h + hp_ref[...]
    u = _layer_norm_relu(h, g_ref[...], be_ref[...])
    logits = jnp.dot(u, Wp_ref[...],
                     preferred_element_type=jnp.float32) + bp_ref[...]
    mx = jnp.max(logits, axis=-1, keepdims=True)
    z = logits - mx
    lse = jnp.log(jnp.sum(jnp.exp(z), axis=-1, keepdims=True))
    out_ref[...] = z - lse


_row_spec = pl.BlockSpec((ROWS, D), lambda i: (i, 0))
_full_spec = pl.BlockSpec((D, D), lambda i: (0, 0))
_vec_spec = pl.BlockSpec((1, D), lambda i: (0, 0))
_P_spec = pl.BlockSpec((2, ROWS, D), lambda i: (0, i, 0))

_fD = jax.ShapeDtypeStruct((N, D), jnp.float32)
_fP = jax.ShapeDtypeStruct((2, N, D), jnp.float32)

_enc_call = pl.pallas_call(
    _enc_body,
    grid=(GRID,),
    in_specs=[_row_spec, _full_spec, _vec_spec],
    out_specs=[_row_spec, _P_spec],
    out_shape=[_fD, _fP],
)

_mid_specs = dict(
    grid=(GRID,),
    in_specs=[_P_spec, _row_spec, _row_spec, _full_spec, _vec_spec,
              _vec_spec, _vec_spec],
    out_specs=[_row_spec, _row_spec, _P_spec],
    out_shape=[_fD, _fD, _fP],
)
_mid_call_nores = pl.pallas_call(functools.partial(_mid_body, False), **_mid_specs)
_mid_call_res = pl.pallas_call(functools.partial(_mid_body, True), **_mid_specs)

_fin_call = pl.pallas_call(
    _fin_body,
    grid=(GRID,),
    in_specs=[_P_spec, _row_spec, _row_spec, _full_spec, _vec_spec,
              _vec_spec, _vec_spec,
              pl.BlockSpec((D, NCLS), lambda i: (0, 0)),
              pl.BlockSpec((1, NCLS), lambda i: (0, 0))],
    out_specs=pl.BlockSpec((ROWS, NCLS), lambda i: (i, 0)),
    out_shape=jax.ShapeDtypeStruct((N, NCLS), jnp.float32),
)


_sc_mesh = plsc.VectorSubcoreMesh(core_axis_name="c", subcore_axis_name="s")


@functools.partial(
    pl.kernel,
    mesh=_sc_mesh,
    out_type=jax.ShapeDtypeStruct((2 * N, D), jnp.float32),
    scratch_types=[
        pltpu.VMEM((SUPER * CHUNK,), jnp.int32),
        pltpu.VMEM((SUPER, CHUNK), jnp.int32),
        pltpu.VMEM((CHUNK, D), jnp.float32),
        pltpu.VMEM((CHUNK, D), jnp.float32),
        pltpu.VMEM((CHUNK, D), jnp.float32),
        pltpu.VMEM((CHUNK, D), jnp.float32),
        pltpu.VMEM_SHARED((N + PADROWS, D), jnp.float32),
        pltpu.SemaphoreType.DMA,
        pltpu.SemaphoreType.DMA,
        pltpu.SemaphoreType.DMA,
        pltpu.SemaphoreType.DMA,
    ],
)
def _spmm_sc(P_hbm, src_hbm, dst_hbm, zeros_hbm, out_hbm,
             src_v, dst_v, rows_a, rows_b, rows_c, rows_d, S_sp,
             sem_a, sem_b, sem_c, sem_d):
    c = lax.axis_index("c")
    s = lax.axis_index("s")

    r0 = s * TROWS
    coff = c * N
    ebase = (c * NS + s) * EDGES_PER_TILE

    @pl.when(s < NS - 1)
    def _zero_main():
        pltpu.sync_copy(zeros_hbm.at[pl.ds(r0, TROWS)],
                        S_sp.at[pl.ds(r0, TROWS)])

    @pl.when(s == NS - 1)
    def _zero_last():
        pltpu.sync_copy(zeros_hbm.at[pl.ds((NS - 1) * TROWS, TROWS_LAST)],
                        S_sp.at[pl.ds((NS - 1) * TROWS, TROWS_LAST)])

    plsc.subcore_barrier()

    def _gather(i, buf, sem):
        pltpu.async_copy(P_hbm.at[src_v.at[pl.ds(i * CHUNK, CHUNK)]],
                         buf, sem)

    def _gwait(i, buf, sem):
        pltpu.make_async_copy(P_hbm.at[src_v.at[pl.ds(i * CHUNK, CHUNK)]],
                              buf, sem).wait()

    def _scat(i, buf):
        pltpu.sync_copy(buf, S_sp.at[dst_v.at[i]], add=True)

    def super_body(u, carry):
        pltpu.sync_copy(
            src_hbm.at[pl.ds(ebase + u * (SUPER * CHUNK), SUPER * CHUNK)],
            src_v)
        pltpu.sync_copy(dst_hbm.at[s, u], dst_v)
        _gather(0, rows_a, sem_a)
        _gather(1, rows_b, sem_b)
        _gather(2, rows_c, sem_c)
        _gather(3, rows_d, sem_d)

        def body(k, carry2):
            i0 = 4 * k
            refire = k < SUPER // 4 - 1
            for d, (buf, sem) in enumerate(
                    [(rows_a, sem_a), (rows_b, sem_b), (rows_c, sem_c),
                     (rows_d, sem_d)]):
                _gwait(i0 + d, buf, sem)
                _scat(i0 + d, buf)

                @pl.when(refire)
                def _next(buf=buf, sem=sem, d=d):
                    _gather(i0 + d + 4, buf, sem)
            return carry2

        lax.fori_loop(0, SUPER // 4, body, 0)
        return carry

    lax.fori_loop(0, NSUPER, super_body, 0)
    plsc.subcore_barrier()

    @pl.when(s < NS - 1)
    def _wb_main():
        pltpu.sync_copy(S_sp.at[pl.ds(r0, TROWS)],
                        out_hbm.at[pl.ds(coff + r0, TROWS)])

    @pl.when(s == NS - 1)
    def _wb_last():
        pltpu.sync_copy(S_sp.at[pl.ds((NS - 1) * TROWS, TROWS_LAST)],
                        out_hbm.at[pl.ds(coff + (NS - 1) * TROWS, TROWS_LAST)])


def kernel(x, edge_index, W_enc, b_enc, W_gcn, b_gcn, gamma, beta,
           W_pred, b_pred):
    src = edge_index[0].astype(jnp.int32)
    dst = edge_index[1].astype(jnp.int32)
    srcp = jnp.concatenate(
        [src, jnp.arange(E_PAD - E, dtype=jnp.int32) % N])
    dstp = jnp.concatenate(
        [dst, N + (jnp.arange(E_PAD - E, dtype=jnp.int32) % PADROWS)])
    src2 = jnp.concatenate([srcp, srcp + N])
    dst3 = dstp.reshape(NS, NSUPER, SUPER, CHUNK)
    zeros = jnp.zeros((N, D), jnp.float32)

    def spmm(P):
        flat = _spmm_sc(P.reshape(2 * N, D), src2, dst3, zeros)
        return flat.reshape(2, N, D)

    b_enc2 = b_enc.reshape(1, D)
    h, P = _enc_call(x, W_enc, b_enc2)
    xp = h
    for r in range(NUM_LAYERS - 1):
        S = spmm(P)
        call = _mid_call_nores if r == 0 else _mid_call_res
        h, xp, P = call(S, xp, h, W_gcn[r], b_gcn[r].reshape(1, D),
                        gamma[r].reshape(1, D), beta[r].reshape(1, D))
    S = spmm(P)
    r = NUM_LAYERS - 1
    return _fin_call(S, xp, h, W_gcn[r], b_gcn[r].reshape(1, D),
                     gamma[r].reshape(1, D), beta[r].reshape(1, D),
                     W_pred, b_pred.reshape(1, NCLS))
